# Initial kernel scaffold; baseline (speedup 1.0000x reference)
import functools

import jax
import jax.numpy as jnp
from jax import lax
from jax.experimental import pallas as pl
from jax.experimental.pallas import tpu as pltpu
from jax.experimental.pallas import tpu_sc as plsc

_NC = 2
_NS = 16
_NW = _NC * _NS


def _sc_segment_sum(table, src_g, dst_g, zeros, n_nodes, d_pad, k_chunk):
    n_chunks = src_g.shape[1]
    rows_per_sub = n_nodes // _NS

    mesh = plsc.VectorSubcoreMesh(
        core_axis_name="c", subcore_axis_name="s",
        num_cores=_NC, num_subcores=_NS)

    @functools.partial(
        pl.kernel,
        mesh=mesh,
        out_type=jax.ShapeDtypeStruct((_NC, n_nodes, d_pad), jnp.float32),
        scratch_types=[
            pltpu.VMEM((n_chunks, k_chunk), jnp.int32),
            pltpu.VMEM((n_chunks, k_chunk), jnp.int32),
            pltpu.VMEM((k_chunk, d_pad), jnp.float32),
            pltpu.VMEM_SHARED((n_nodes, d_pad), jnp.float32),
            pltpu.SemaphoreType.DMA,
        ],
    )
    def seg_kernel(table_hbm, src_hbm, dst_hbm, zeros_hbm, out_hbm,
                   src_v, dst_v, rows_v, acc_sh, sem):
        c = lax.axis_index("c")
        s = lax.axis_index("s")
        wid = s * _NC + c

        pltpu.sync_copy(zeros_hbm.at[pl.ds(s * rows_per_sub, rows_per_sub)],
                        acc_sh.at[pl.ds(s * rows_per_sub, rows_per_sub)])
        pltpu.sync_copy(src_hbm.at[wid], src_v)
        pltpu.sync_copy(dst_hbm.at[wid], dst_v)
        plsc.subcore_barrier()

        def body(i, carry):
            pltpu.async_copy(table_hbm.at[src_v.at[i]], rows_v, sem).wait()
            pltpu.sync_copy(rows_v, acc_sh.at[dst_v.at[i]], add=True)
            return carry

        lax.fori_loop(0, n_chunks, body, 0)

        plsc.subcore_barrier()
        pltpu.sync_copy(acc_sh.at[pl.ds(s * rows_per_sub, rows_per_sub)],
                        out_hbm.at[c, pl.ds(s * rows_per_sub, rows_per_sub)])

    return seg_kernel(table, src_g, dst_g, zeros)


def _tc_stage1(x, acc1, Wl1, bl1, Wr1, W1, b1, Wl2, Wr2, bl2, blk):
    n = x.shape[0]
    d = x.shape[1]
    cdim = Wl2.shape[0]

    def body(x_ref, acc_ref, wl1_ref, bl1_ref, wr1_ref, w1_ref, b1_ref,
             wl2_ref, wr2_ref, bl2_ref, y2p_ref, z2c_ref):
        acc = acc_ref[0] + acc_ref[1]
        agg_sum = acc[:, :d]
        cnt = jnp.maximum(acc[:, d:d + 1], 1.0)
        agg = agg_sum / cnt
        xb = x_ref[...]

        dot = lambda a, w: lax.dot_general(
            a, w, (((1,), (1,)), ((), ())), preferred_element_type=jnp.float32)

        pre1 = dot(agg, wl1_ref[...]) + bl1_ref[...] + dot(xb, wr1_ref[...])
        nrm1 = jnp.sqrt(jnp.sum(pre1 * pre1, axis=1, keepdims=True))
        h1 = jnp.maximum(pre1 / jnp.maximum(nrm1, 1e-12), 0.0)

        w1 = w1_ref[...]
        h2 = jnp.maximum(dot(xb, w1[:, :d]) + dot(h1, w1[:, d:]) + b1_ref[...],
                         0.0)

        y2 = dot(h2, wl2_ref[...])
        bsz = y2.shape[0]
        y2p_ref[...] = jnp.concatenate(
            [y2, jnp.zeros((bsz, 64 - cdim), jnp.float32)], axis=1)
        z2 = dot(h2, wr2_ref[...]) + bl2_ref[...]
        z2c_ref[...] = jnp.concatenate(
            [z2, cnt, jnp.zeros((bsz, 128 - cdim - 1), jnp.float32)], axis=1)

    d_pad = acc1.shape[-1]
    grid = (n // blk,)
    wspec = lambda shp: pl.BlockSpec(shp, lambda i: (0,) * len(shp))
    return pl.pallas_call(
        body,
        grid=grid,
        in_specs=[
            pl.BlockSpec((blk, d), lambda i: (i, 0)),
            pl.BlockSpec((_NC, blk, d_pad), lambda i: (0, i, 0)),
            wspec(Wl1.shape), wspec(bl1.shape), wspec(Wr1.shape),
            wspec(W1.shape), wspec(b1.shape), wspec(Wl2.shape),
            wspec(Wr2.shape), wspec(bl2.shape),
        ],
        out_specs=[
            pl.BlockSpec((blk, 64), lambda i: (i, 0)),
            pl.BlockSpec((blk, 128), lambda i: (i, 0)),
        ],
        out_shape=[
            jax.ShapeDtypeStruct((n, 64), jnp.float32),
            jax.ShapeDtypeStruct((n, 128), jnp.float32),
        ],
    )(x, acc1, Wl1, bl1, Wr1, W1, b1, Wl2, Wr2, bl2)


def _tc_stage2(acc2, z2c, cdim, blk):
    n = z2c.shape[0]

    def body(acc_ref, z2c_ref, out_ref):
        acc = acc_ref[0] + acc_ref[1]
        agg_sum = acc[:, :cdim]
        z2cb = z2c_ref[...]
        z2 = z2cb[:, :cdim]
        cnt = z2cb[:, cdim:cdim + 1]
        pre = agg_sum / cnt + z2
        nrm = jnp.sqrt(jnp.sum(pre * pre, axis=1, keepdims=True))
        out_ref[...] = pre / jnp.maximum(nrm, 1e-12)

    grid = (n // blk,)
    return pl.pallas_call(
        body,
        grid=grid,
        in_specs=[
            pl.BlockSpec((_NC, blk, 64), lambda i: (0, i, 0)),
            pl.BlockSpec((blk, 128), lambda i: (i, 0)),
        ],
        out_specs=pl.BlockSpec((blk, cdim), lambda i: (i, 0)),
        out_shape=jax.ShapeDtypeStruct((n, cdim), jnp.float32),
    )(acc2, z2c)


def kernel(x, Wl1, bl1, Wr1, W1, b1, Wl2, bl2, Wr2, edge_index):
    n, d = x.shape
    e = edge_index.shape[1]
    cdim = Wl2.shape[0]

    d1_pad = 144
    k_chunk = 80
    epw = e // _NW
    n_chunks = epw // k_chunk

    src_g = edge_index[0].reshape(_NW, n_chunks, k_chunk)
    dst_g = edge_index[1].reshape(_NW, n_chunks, k_chunk)

    table1 = jnp.concatenate(
        [x, jnp.ones((n, 1), jnp.float32),
         jnp.zeros((n, d1_pad - d - 1), jnp.float32)], axis=1)
    zeros1 = jnp.zeros((n, d1_pad), jnp.float32)

    acc1 = _sc_segment_sum(table1, src_g, dst_g, zeros1, n, d1_pad, k_chunk)

    bl1r = bl1.reshape(1, -1)
    b1r = b1.reshape(1, -1)
    bl2r = bl2.reshape(1, -1)
    y2p, z2c = _tc_stage1(x, acc1, Wl1, bl1r, Wr1, W1, b1r, Wl2, Wr2, bl2r,
                          blk=2000)

    zeros2 = jnp.zeros((n, 64), jnp.float32)
    acc2 = _sc_segment_sum(y2p, src_g, dst_g, zeros2, n, 64, k_chunk)

    return _tc_stage2(acc2, z2c, cdim, blk=2000)


# speedup vs baseline: 7.2620x; 7.2620x over previous
import functools

import jax
import jax.numpy as jnp
from jax import lax
from jax.experimental import pallas as pl
from jax.experimental.pallas import tpu as pltpu
from jax.experimental.pallas import tpu_sc as plsc

_NC = 2
_NS = 16
_NW = _NC * _NS


def _sc_segment_sum(table, src_g, dst_g, zeros, n_nodes, d_pad, k_chunk):
    n_chunks = src_g.shape[1]
    rows_per_sub = n_nodes // _NS

    mesh = plsc.VectorSubcoreMesh(
        core_axis_name="c", subcore_axis_name="s",
        num_cores=_NC, num_subcores=_NS)

    @functools.partial(
        pl.kernel,
        mesh=mesh,
        out_type=jax.ShapeDtypeStruct((_NC, n_nodes, d_pad), jnp.float32),
        scratch_types=[
            pltpu.VMEM((n_chunks, k_chunk), jnp.int32),
            pltpu.VMEM((n_chunks, k_chunk), jnp.int32),
            pltpu.VMEM((k_chunk, d_pad), jnp.float32),
            pltpu.VMEM_SHARED((n_nodes, d_pad), jnp.float32),
            pltpu.SemaphoreType.DMA,
        ],
        compiler_params=pltpu.CompilerParams(use_tc_tiling_on_sc=False),
    )
    def seg_kernel(table_hbm, src_hbm, dst_hbm, zeros_hbm, out_hbm,
                   src_v, dst_v, rows_v, acc_sh, sem):
        c = lax.axis_index("c")
        s = lax.axis_index("s")
        wid = s * _NC + c

        pltpu.sync_copy(zeros_hbm.at[pl.ds(s * rows_per_sub, rows_per_sub)],
                        acc_sh.at[pl.ds(s * rows_per_sub, rows_per_sub)])
        pltpu.sync_copy(src_hbm.at[wid], src_v)
        pltpu.sync_copy(dst_hbm.at[wid], dst_v)
        plsc.subcore_barrier()

        def body(i, carry):
            pltpu.async_copy(table_hbm.at[src_v.at[i]], rows_v, sem).wait()
            pltpu.sync_copy(rows_v, acc_sh.at[dst_v.at[i]], add=True)
            return carry

        lax.fori_loop(0, n_chunks, body, 0)

        plsc.subcore_barrier()
        pltpu.sync_copy(acc_sh.at[pl.ds(s * rows_per_sub, rows_per_sub)],
                        out_hbm.at[c, pl.ds(s * rows_per_sub, rows_per_sub)])

    return seg_kernel(table, src_g, dst_g, zeros)


def _tc_stage1(x, acc1, Wl1, bl1, Wr1, W1, b1, Wl2, Wr2, bl2, blk):
    n = x.shape[0]
    d = x.shape[1]
    cdim = Wl2.shape[0]

    def body(x_ref, acc_ref, wl1_ref, bl1_ref, wr1_ref, w1_ref, b1_ref,
             wl2_ref, wr2_ref, bl2_ref, y2p_ref, z2c_ref):
        acc = acc_ref[0] + acc_ref[1]
        agg_sum = acc[:, :d]
        cnt = jnp.maximum(acc[:, d:d + 1], 1.0)
        agg = agg_sum / cnt
        xb = x_ref[...]

        dot = lambda a, w: lax.dot_general(
            a, w, (((1,), (1,)), ((), ())), preferred_element_type=jnp.float32)

        pre1 = dot(agg, wl1_ref[...]) + bl1_ref[...] + dot(xb, wr1_ref[...])
        nrm1 = jnp.sqrt(jnp.sum(pre1 * pre1, axis=1, keepdims=True))
        h1 = jnp.maximum(pre1 / jnp.maximum(nrm1, 1e-12), 0.0)

        w1 = w1_ref[...]
        h2 = jnp.maximum(dot(xb, w1[:, :d]) + dot(h1, w1[:, d:]) + b1_ref[...],
                         0.0)

        y2 = dot(h2, wl2_ref[...])
        bsz = y2.shape[0]
        y2p_ref[...] = jnp.concatenate(
            [y2, jnp.zeros((bsz, 64 - cdim), jnp.float32)], axis=1)
        z2 = dot(h2, wr2_ref[...]) + bl2_ref[...]
        z2c_ref[...] = jnp.concatenate(
            [z2, cnt, jnp.zeros((bsz, 128 - cdim - 1), jnp.float32)], axis=1)

    d_pad = acc1.shape[-1]
    grid = (n // blk,)
    wspec = lambda shp: pl.BlockSpec(shp, lambda i: (0,) * len(shp))
    return pl.pallas_call(
        body,
        grid=grid,
        in_specs=[
            pl.BlockSpec((blk, d), lambda i: (i, 0)),
            pl.BlockSpec((_NC, blk, d_pad), lambda i: (0, i, 0)),
            wspec(Wl1.shape), wspec(bl1.shape), wspec(Wr1.shape),
            wspec(W1.shape), wspec(b1.shape), wspec(Wl2.shape),
            wspec(Wr2.shape), wspec(bl2.shape),
        ],
        out_specs=[
            pl.BlockSpec((blk, 64), lambda i: (i, 0)),
            pl.BlockSpec((blk, 128), lambda i: (i, 0)),
        ],
        out_shape=[
            jax.ShapeDtypeStruct((n, 64), jnp.float32),
            jax.ShapeDtypeStruct((n, 128), jnp.float32),
        ],
    )(x, acc1, Wl1, bl1, Wr1, W1, b1, Wl2, Wr2, bl2)


def _tc_stage2(acc2, z2c, cdim, blk):
    n = z2c.shape[0]

    def body(acc_ref, z2c_ref, out_ref):
        acc = acc_ref[0] + acc_ref[1]
        agg_sum = acc[:, :cdim]
        z2cb = z2c_ref[...]
        z2 = z2cb[:, :cdim]
        cnt = z2cb[:, cdim:cdim + 1]
        pre = agg_sum / cnt + z2
        nrm = jnp.sqrt(jnp.sum(pre * pre, axis=1, keepdims=True))
        out_ref[...] = pre / jnp.maximum(nrm, 1e-12)

    grid = (n // blk,)
    return pl.pallas_call(
        body,
        grid=grid,
        in_specs=[
            pl.BlockSpec((_NC, blk, 64), lambda i: (0, i, 0)),
            pl.BlockSpec((blk, 128), lambda i: (i, 0)),
        ],
        out_specs=pl.BlockSpec((blk, cdim), lambda i: (i, 0)),
        out_shape=jax.ShapeDtypeStruct((n, cdim), jnp.float32),
    )(acc2, z2c)


def kernel(x, Wl1, bl1, Wr1, W1, b1, Wl2, bl2, Wr2, edge_index):
    n, d = x.shape
    e = edge_index.shape[1]
    cdim = Wl2.shape[0]

    d1_pad = 144
    k_chunk = 80
    epw = e // _NW
    n_chunks = epw // k_chunk

    src_g = edge_index[0].reshape(_NW, n_chunks, k_chunk)
    dst_g = edge_index[1].reshape(_NW, n_chunks, k_chunk)

    table1 = jnp.concatenate(
        [x, jnp.ones((n, 1), jnp.float32),
         jnp.zeros((n, d1_pad - d - 1), jnp.float32)], axis=1)
    zeros1 = jnp.zeros((n, d1_pad), jnp.float32)

    acc1 = _sc_segment_sum(table1, src_g, dst_g, zeros1, n, d1_pad, k_chunk)

    bl1r = bl1.reshape(1, -1)
    b1r = b1.reshape(1, -1)
    bl2r = bl2.reshape(1, -1)
    y2p, z2c = _tc_stage1(x, acc1, Wl1, bl1r, Wr1, W1, b1r, Wl2, Wr2, bl2r,
                          blk=2000)

    zeros2 = jnp.zeros((n, 64), jnp.float32)
    acc2 = _sc_segment_sum(y2p, src_g, dst_g, zeros2, n, 64, k_chunk)

    return _tc_stage2(acc2, z2c, cdim, blk=2000)
